# baseline (speedup 1.0000x reference)
import functools

import jax
import jax.numpy as jnp
from jax import lax
from jax.experimental import pallas as pl
from jax.experimental.pallas import tpu as pltpu
from jax.experimental.pallas import tpu_sc as plsc

N = 10000
E = 320000
D_IN = 128
D_HID = 128
D_OUT = 64

NC = 2
NS = 16
NT = NC * NS
EPT = E // NT
CH = 40
NCH = EPT // CH
NB = 5
NRND = NCH // NB
RPT = N // NS
ZR = 125


ZCH = 16
NZCH = N // ZCH
RCH = 2000
NRCH = N // RCH


def _sc_agg_body(D, *refs):
    (x_hbm, src_hbm, dst2_hbm, out_hbm,
     acc, src_v, dst_v, rows_v, zbuf, gsem, ssem) = refs

    c = lax.axis_index("c")
    s = lax.axis_index("s")
    wid = s * NC + c

    zvec = jnp.zeros((16,), jnp.float32)

    def zrow(r, carry):
        def zcol(k, carry2):
            zbuf[r, pl.ds(k * 16, 16)] = zvec
            return carry2
        return lax.fori_loop(0, D // 16, zcol, carry)

    lax.fori_loop(0, ZCH, zrow, 0)

    def zchunk(k, carry):
        chunk = s + NS * k

        @pl.when(chunk < NZCH)
        def _():
            pltpu.sync_copy(zbuf, acc.at[pl.ds(chunk * ZCH, ZCH)])
        return carry

    lax.fori_loop(0, -(-NZCH // NS), zchunk, 0)

    plsc.subcore_barrier()

    ebase = wid * EPT
    pltpu.sync_copy(src_hbm.at[pl.ds(ebase, EPT)], src_v)
    pltpu.sync_copy(dst2_hbm.at[wid], dst_v)

    def g_desc(j, b):
        return pltpu.make_async_copy(
            x_hbm.at[src_v.at[pl.ds(j * CH, CH)]], rows_v.at[b], gsem.at[b])

    def s_desc(j, b):
        return pltpu.make_async_copy(
            rows_v.at[b], acc.at[dst_v.at[j]], ssem.at[b])

    for b in range(NB):
        g_desc(b, b).start()

    def rnd(i, carry):
        j0 = i * NB
        for b in range(NB):
            g_desc(j0 + b, b).wait()
            s_desc(j0 + b, b).start(add=True)
        for b in range(NB):
            s_desc(j0 + b, b).wait()

            @pl.when(i + 1 < NRND)
            def _():
                g_desc(j0 + NB + b, b).start()
        return carry

    lax.fori_loop(0, NRND, rnd, 0)

    plsc.subcore_barrier()

    @pl.when(s < NRCH)
    def _():
        r0 = s * RCH
        pltpu.sync_copy(acc.at[pl.ds(r0, RCH)],
                        out_hbm.at[pl.ds(c * N + r0, RCH)])


def _make_sc_agg(D):
    mesh = plsc.VectorSubcoreMesh(core_axis_name="c", subcore_axis_name="s")
    scratch = [
        pltpu.VMEM_SHARED((N, D), jnp.float32),
        pltpu.VMEM((EPT,), jnp.int32),
        pltpu.VMEM((NCH, CH), jnp.int32),
        pltpu.VMEM((NB, CH, D), jnp.float32),
        pltpu.VMEM((ZCH, D), jnp.float32),
        pltpu.SemaphoreType.DMA((NB,)),
        pltpu.SemaphoreType.DMA((NB,)),
    ]
    return pl.kernel(
        functools.partial(_sc_agg_body, D),
        out_type=jax.ShapeDtypeStruct((NC * N, D), jnp.float32),
        mesh=mesh,
        scratch_types=scratch,
        compiler_params=pltpu.CompilerParams(use_tc_tiling_on_sc=False),
    )


_sc_agg_64 = _make_sc_agg(D_OUT)

RND_E = NB * CH


def _sc_agg128_cnt_body(x_hbm, src_hbm, dst2_hbm, z128_hbm, z16_hbm,
                        out_hbm, cnt_hbm,
                        acc, cnt_sh, srcb0, srcb1, dst_v, rows_v, ones_v,
                        gsem, ssem, csem, srcsem0, srcsem1):
    D = D_IN
    c = lax.axis_index("c")
    s = lax.axis_index("s")
    wid = s * NC + c

    ovec = jnp.ones((16,), jnp.float32)

    def orow(r, carry):
        ones_v[r, pl.ds(0, 16)] = ovec
        return carry
    lax.fori_loop(0, CH, orow, 0)

    @pl.when(s < NRCH)
    def _():
        r0 = s * RCH
        pltpu.sync_copy(z128_hbm, acc.at[pl.ds(r0, RCH)])
        pltpu.sync_copy(z16_hbm, cnt_sh.at[pl.ds(r0, RCH)])

    plsc.subcore_barrier()

    ebase = wid * EPT
    pltpu.sync_copy(dst2_hbm.at[wid], dst_v)

    srcbs = (srcb0, srcb1)
    srcsems = (srcsem0, srcsem1)

    def srcload(r, par):
        return pltpu.make_async_copy(
            src_hbm.at[pl.ds(ebase + r * RND_E, RND_E)], srcbs[par],
            srcsems[par])

    def g_desc(j, b, par):
        return pltpu.make_async_copy(
            x_hbm.at[srcbs[par].at[pl.ds(b * CH, CH)]], rows_v.at[b],
            gsem.at[b])

    def s_desc(j, b):
        return pltpu.make_async_copy(
            rows_v.at[b], acc.at[dst_v.at[j]], ssem.at[b])

    def o_desc(j, b):
        return pltpu.make_async_copy(
            ones_v, cnt_sh.at[dst_v.at[j]], csem.at[b])

    srcload(0, 0).start()
    srcload(0, 0).wait()
    srcload(1, 1).start()
    for b in range(NB):
        g_desc(b, b, 0).start()

    def rnd(i, par):
        j0 = i * NB
        for b in range(NB):
            j = j0 + b
            g_desc(j, b, par).wait()
            s_desc(j, b).start(add=True)
            o_desc(j, b).start(add=True)

        @pl.when(i + 2 < NRND)
        def _():
            srcload(i + 2, par).start()

        @pl.when(i + 1 < NRND)
        def _():
            srcload(i + 1, 1 - par).wait()

        for b in range(NB):
            j = j0 + b
            s_desc(j, b).wait()
            o_desc(j, b).wait()

            @pl.when(i + 1 < NRND)
            def _():
                g_desc(j + NB, b, 1 - par).start()

    def rnd_pair(p, carry):
        rnd(2 * p, 0)
        rnd(2 * p + 1, 1)
        return carry

    lax.fori_loop(0, NRND // 2, rnd_pair, 0)

    plsc.subcore_barrier()

    @pl.when(s < NRCH)
    def _():
        r0 = s * RCH
        pltpu.sync_copy(acc.at[pl.ds(r0, RCH)],
                        out_hbm.at[pl.ds(c * N + r0, RCH)])
        pltpu.sync_copy(cnt_sh.at[pl.ds(r0, RCH)],
                        cnt_hbm.at[pl.ds(c * N + r0, RCH)])


_sc_agg128_cnt = pl.kernel(
    _sc_agg128_cnt_body,
    out_type=[jax.ShapeDtypeStruct((NC * N, D_IN), jnp.float32),
              jax.ShapeDtypeStruct((NC * N, 16), jnp.float32)],
    mesh=plsc.VectorSubcoreMesh(core_axis_name="c", subcore_axis_name="s"),
    scratch_types=[
        pltpu.VMEM_SHARED((N, D_IN), jnp.float32),
        pltpu.VMEM_SHARED((N, 16), jnp.float32),
        pltpu.VMEM((RND_E,), jnp.int32),
        pltpu.VMEM((RND_E,), jnp.int32),
        pltpu.VMEM((NCH, CH), jnp.int32),
        pltpu.VMEM((NB, CH, D_IN), jnp.float32),
        pltpu.VMEM((CH, 16), jnp.float32),
        pltpu.SemaphoreType.DMA((NB,)),
        pltpu.SemaphoreType.DMA((NB,)),
        pltpu.SemaphoreType.DMA((NB,)),
        pltpu.SemaphoreType.DMA,
        pltpu.SemaphoreType.DMA,
    ],
    compiler_params=pltpu.CompilerParams(use_tc_tiling_on_sc=False),
)


TR = 1000
TG = N // TR


def _tc1_body(xb_ref, s1a_ref, s1b_ref, cnt_ref, wl1_ref, bl1_ref, wr1_ref,
              wl2_ref, h_ref, y2_ref):
    i = pl.program_id(0)
    r0 = i * TR
    cblk = cnt_ref[pl.ds(r0, TR), :] + cnt_ref[pl.ds(N + r0, TR), :]
    cnt = jnp.sum(cblk, axis=1) * (1.0 / 16.0)
    ssum = s1a_ref[...] + s1b_ref[...]
    mean = ssum / jnp.maximum(cnt, 1.0)[:, None]
    dn = (((1,), (1,)), ((), ()))
    h = (lax.dot_general(mean, wl1_ref[...], dn,
                         preferred_element_type=jnp.float32)
         + bl1_ref[...]
         + lax.dot_general(xb_ref[...], wr1_ref[...], dn,
                           preferred_element_type=jnp.float32))
    h = jnp.maximum(h, 0.0)
    h_ref[...] = h
    y2_ref[pl.ds(r0, TR), :] = lax.dot_general(
        h, wl2_ref[...], dn, preferred_element_type=jnp.float32)


def _tc2_body(s2_ref, cnt_ref, h_ref, wr2_ref, bl2_ref, o_ref):
    i = pl.program_id(0)
    r0 = i * TR
    cblk = cnt_ref[pl.ds(r0, TR), :] + cnt_ref[pl.ds(N + r0, TR), :]
    cnt = jnp.sum(cblk, axis=1) * (1.0 / 16.0)
    m2 = ((s2_ref[pl.ds(r0, TR), :] + s2_ref[pl.ds(N + r0, TR), :])
          / jnp.maximum(cnt, 1.0)[:, None])
    dn = (((1,), (1,)), ((), ()))
    o_ref[pl.ds(r0, TR), :] = (
        m2 + bl2_ref[...]
        + lax.dot_general(h_ref[...], wr2_ref[...], dn,
                          preferred_element_type=jnp.float32))


def _whole(shape):
    return pl.BlockSpec(shape, lambda i: (0,) * len(shape))


_tc1 = pl.pallas_call(
    _tc1_body,
    grid=(TG,),
    in_specs=[
        pl.BlockSpec((TR, D_IN), lambda i: (i, 0)),
        pl.BlockSpec((TR, D_IN), lambda i: (i, 0)),
        pl.BlockSpec((TR, D_IN), lambda i: (i + TG, 0)),
        _whole((NC * N, 16)),
        _whole((D_HID, D_IN)),
        _whole((1, D_HID)),
        _whole((D_HID, D_IN)),
        _whole((D_OUT, D_HID)),
    ],
    out_specs=[
        pl.BlockSpec((TR, D_HID), lambda i: (i, 0)),
        _whole((N, D_OUT)),
    ],
    out_shape=[jax.ShapeDtypeStruct((N, D_HID), jnp.float32),
               jax.ShapeDtypeStruct((N, D_OUT), jnp.float32)],
)

_tc2 = pl.pallas_call(
    _tc2_body,
    grid=(TG,),
    in_specs=[
        _whole((NC * N, D_OUT)),
        _whole((NC * N, 16)),
        pl.BlockSpec((TR, D_HID), lambda i: (i, 0)),
        _whole((D_OUT, D_HID)),
        _whole((1, D_OUT)),
    ],
    out_specs=_whole((N, D_OUT)),
    out_shape=jax.ShapeDtypeStruct((N, D_OUT), jnp.float32),
)


def kernel(x, edge_index, Wl1, bl1, Wr1, Wl2, bl2, Wr2):
    src = edge_index[0]
    dst = edge_index[1]
    dst2 = dst.reshape(NT, NCH, CH)

    z128 = jnp.zeros((RCH, D_IN), jnp.float32)
    z16 = jnp.zeros((RCH, 16), jnp.float32)
    s1, cnt = _sc_agg128_cnt(x, src, dst2, z128, z16)
    h, y2 = _tc1(x, s1, s1, cnt, Wl1, bl1.reshape(1, D_HID), Wr1, Wl2)
    s2 = _sc_agg_64(y2, src, dst2)
    out = _tc2(s2, cnt, h, Wr2, bl2.reshape(1, D_OUT))
    return out


# speedup vs baseline: 1.0200x; 1.0200x over previous
import functools

import jax
import jax.numpy as jnp
from jax import lax
from jax.experimental import pallas as pl
from jax.experimental.pallas import tpu as pltpu
from jax.experimental.pallas import tpu_sc as plsc

N = 10000
E = 320000
D_IN = 128
D_HID = 128
D_OUT = 64

NC = 2
NS = 16
NT = NC * NS
EPT = E // NT
CH = 40
NCH = EPT // CH
NB = 5
NRND = NCH // NB
RPT = N // NS
ZR = 125


ZCH = 16
NZCH = N // ZCH
RCH = 2000
NRCH = N // RCH


def _sc_agg_body(D, *refs):
    (x_hbm, src_hbm, dst2_hbm, out_hbm,
     acc, src_v, dst_v, rows_v, zbuf, gsem, ssem) = refs

    c = lax.axis_index("c")
    s = lax.axis_index("s")
    wid = s * NC + c

    zvec = jnp.zeros((16,), jnp.float32)

    def zrow(r, carry):
        def zcol(k, carry2):
            zbuf[r, pl.ds(k * 16, 16)] = zvec
            return carry2
        return lax.fori_loop(0, D // 16, zcol, carry)

    lax.fori_loop(0, ZCH, zrow, 0)

    def zchunk(k, carry):
        chunk = s + NS * k

        @pl.when(chunk < NZCH)
        def _():
            pltpu.sync_copy(zbuf, acc.at[pl.ds(chunk * ZCH, ZCH)])
        return carry

    lax.fori_loop(0, -(-NZCH // NS), zchunk, 0)

    plsc.subcore_barrier()

    ebase = wid * EPT
    pltpu.sync_copy(src_hbm.at[pl.ds(ebase, EPT)], src_v)
    pltpu.sync_copy(dst2_hbm.at[wid], dst_v)

    def g_desc(j, b):
        return pltpu.make_async_copy(
            x_hbm.at[src_v.at[pl.ds(j * CH, CH)]], rows_v.at[b], gsem.at[b])

    def s_desc(j, b):
        return pltpu.make_async_copy(
            rows_v.at[b], acc.at[dst_v.at[j]], ssem.at[b])

    for b in range(NB):
        g_desc(b, b).start()

    def rnd(i, carry):
        j0 = i * NB
        for b in range(NB):
            g_desc(j0 + b, b).wait()
            s_desc(j0 + b, b).start(add=True)
        for b in range(NB):
            s_desc(j0 + b, b).wait()

            @pl.when(i + 1 < NRND)
            def _():
                g_desc(j0 + NB + b, b).start()
        return carry

    lax.fori_loop(0, NRND, rnd, 0)

    plsc.subcore_barrier()

    @pl.when(s < NRCH)
    def _():
        r0 = s * RCH
        pltpu.sync_copy(acc.at[pl.ds(r0, RCH)],
                        out_hbm.at[pl.ds(c * N + r0, RCH)])


def _make_sc_agg(D):
    mesh = plsc.VectorSubcoreMesh(core_axis_name="c", subcore_axis_name="s")
    scratch = [
        pltpu.VMEM_SHARED((N, D), jnp.float32),
        pltpu.VMEM((EPT,), jnp.int32),
        pltpu.VMEM((NCH, CH), jnp.int32),
        pltpu.VMEM((NB, CH, D), jnp.float32),
        pltpu.VMEM((ZCH, D), jnp.float32),
        pltpu.SemaphoreType.DMA((NB,)),
        pltpu.SemaphoreType.DMA((NB,)),
    ]
    return pl.kernel(
        functools.partial(_sc_agg_body, D),
        out_type=jax.ShapeDtypeStruct((NC * N, D), jnp.float32),
        mesh=mesh,
        scratch_types=scratch,
        compiler_params=pltpu.CompilerParams(use_tc_tiling_on_sc=False),
    )


_sc_agg_64 = _make_sc_agg(D_OUT)

RND_E = NB * CH


def _sc_agg128_cnt_body(x_hbm, src_hbm, dst2_hbm, z128_hbm, z16_hbm,
                        out_hbm, cnt_hbm,
                        acc, cnt_sh, srcb0, srcb1, dst_v, rows_v, ones_v,
                        gsem, ssem, csem, srcsem0, srcsem1):
    D = D_IN
    c = lax.axis_index("c")
    s = lax.axis_index("s")
    wid = s * NC + c

    ovec = jnp.ones((16,), jnp.float32)

    def orow(r, carry):
        ones_v[r, pl.ds(0, 16)] = ovec
        return carry
    lax.fori_loop(0, CH, orow, 0)

    @pl.when(s < NRCH)
    def _():
        r0 = s * RCH
        pltpu.sync_copy(z128_hbm, acc.at[pl.ds(r0, RCH)])
        pltpu.sync_copy(z16_hbm, cnt_sh.at[pl.ds(r0, RCH)])

    plsc.subcore_barrier()

    ebase = wid * EPT
    pltpu.sync_copy(dst2_hbm.at[wid], dst_v)

    srcbs = (srcb0, srcb1)
    srcsems = (srcsem0, srcsem1)

    def srcload(r, par):
        return pltpu.make_async_copy(
            src_hbm.at[pl.ds(ebase + r * RND_E, RND_E)], srcbs[par],
            srcsems[par])

    def g_desc(j, b, par):
        return pltpu.make_async_copy(
            x_hbm.at[srcbs[par].at[pl.ds(b * CH, CH)]], rows_v.at[b],
            gsem.at[b])

    def s_desc(j, b):
        return pltpu.make_async_copy(
            rows_v.at[b], acc.at[dst_v.at[j]], ssem.at[b])

    def o_desc(j, b):
        return pltpu.make_async_copy(
            ones_v, cnt_sh.at[dst_v.at[j]], csem.at[b])

    srcload(0, 0).start()
    srcload(0, 0).wait()
    srcload(1, 1).start()
    for b in range(NB):
        g_desc(b, b, 0).start()

    def rnd(i, par):
        j0 = i * NB
        for b in range(NB):
            j = j0 + b
            g_desc(j, b, par).wait()
            s_desc(j, b).start(add=True)
            o_desc(j, b).start(add=True)

        @pl.when(i + 2 < NRND)
        def _():
            srcload(i + 2, par).start()

        @pl.when(i + 1 < NRND)
        def _():
            srcload(i + 1, 1 - par).wait()

        for b in range(NB):
            j = j0 + b
            s_desc(j, b).wait()
            o_desc(j, b).wait()

            @pl.when(i + 1 < NRND)
            def _():
                g_desc(j + NB, b, 1 - par).start()

    def rnd_pair(p, carry):
        rnd(2 * p, 0)
        rnd(2 * p + 1, 1)
        return carry

    lax.fori_loop(0, NRND // 2, rnd_pair, 0)

    plsc.subcore_barrier()

    @pl.when(s < NRCH)
    def _():
        r0 = s * RCH
        pltpu.sync_copy(acc.at[pl.ds(r0, RCH)],
                        out_hbm.at[pl.ds(c * N + r0, RCH)])
        pltpu.sync_copy(cnt_sh.at[pl.ds(r0, RCH)],
                        cnt_hbm.at[pl.ds(c * N + r0, RCH)])


_sc_agg128_cnt = pl.kernel(
    _sc_agg128_cnt_body,
    out_type=[jax.ShapeDtypeStruct((NC * N, D_IN), jnp.float32),
              jax.ShapeDtypeStruct((NC * N, 16), jnp.float32)],
    mesh=plsc.VectorSubcoreMesh(core_axis_name="c", subcore_axis_name="s"),
    scratch_types=[
        pltpu.VMEM_SHARED((N, D_IN), jnp.float32),
        pltpu.VMEM_SHARED((N, 16), jnp.float32),
        pltpu.VMEM((RND_E,), jnp.int32),
        pltpu.VMEM((RND_E,), jnp.int32),
        pltpu.VMEM((NCH, CH), jnp.int32),
        pltpu.VMEM((NB, CH, D_IN), jnp.float32),
        pltpu.VMEM((CH, 16), jnp.float32),
        pltpu.SemaphoreType.DMA((NB,)),
        pltpu.SemaphoreType.DMA((NB,)),
        pltpu.SemaphoreType.DMA((NB,)),
        pltpu.SemaphoreType.DMA,
        pltpu.SemaphoreType.DMA,
    ],
    compiler_params=pltpu.CompilerParams(use_tc_tiling_on_sc=False),
)


TR = 2000
TG = N // TR


def _tc1_body(xb_ref, s1a_ref, s1b_ref, cnt_ref, wl1_ref, bl1_ref, wr1_ref,
              wl2_ref, h_ref, y2_ref):
    i = pl.program_id(0)
    r0 = i * TR
    cblk = cnt_ref[pl.ds(r0, TR), :] + cnt_ref[pl.ds(N + r0, TR), :]
    cnt = jnp.sum(cblk, axis=1) * (1.0 / 16.0)
    ssum = s1a_ref[...] + s1b_ref[...]
    mean = ssum / jnp.maximum(cnt, 1.0)[:, None]
    dn = (((1,), (1,)), ((), ()))
    h = (lax.dot_general(mean, wl1_ref[...], dn,
                         preferred_element_type=jnp.float32)
         + bl1_ref[...]
         + lax.dot_general(xb_ref[...], wr1_ref[...], dn,
                           preferred_element_type=jnp.float32))
    h = jnp.maximum(h, 0.0)
    h_ref[...] = h
    y2_ref[pl.ds(r0, TR), :] = lax.dot_general(
        h, wl2_ref[...], dn, preferred_element_type=jnp.float32)


def _tc2_body(s2_ref, cnt_ref, h_ref, wr2_ref, bl2_ref, o_ref):
    i = pl.program_id(0)
    r0 = i * TR
    cblk = cnt_ref[pl.ds(r0, TR), :] + cnt_ref[pl.ds(N + r0, TR), :]
    cnt = jnp.sum(cblk, axis=1) * (1.0 / 16.0)
    m2 = ((s2_ref[pl.ds(r0, TR), :] + s2_ref[pl.ds(N + r0, TR), :])
          / jnp.maximum(cnt, 1.0)[:, None])
    dn = (((1,), (1,)), ((), ()))
    o_ref[pl.ds(r0, TR), :] = (
        m2 + bl2_ref[...]
        + lax.dot_general(h_ref[...], wr2_ref[...], dn,
                          preferred_element_type=jnp.float32))


def _whole(shape):
    return pl.BlockSpec(shape, lambda i: (0,) * len(shape))


_tc1 = pl.pallas_call(
    _tc1_body,
    grid=(TG,),
    in_specs=[
        pl.BlockSpec((TR, D_IN), lambda i: (i, 0)),
        pl.BlockSpec((TR, D_IN), lambda i: (i, 0)),
        pl.BlockSpec((TR, D_IN), lambda i: (i + TG, 0)),
        _whole((NC * N, 16)),
        _whole((D_HID, D_IN)),
        _whole((1, D_HID)),
        _whole((D_HID, D_IN)),
        _whole((D_OUT, D_HID)),
    ],
    out_specs=[
        pl.BlockSpec((TR, D_HID), lambda i: (i, 0)),
        _whole((N, D_OUT)),
    ],
    out_shape=[jax.ShapeDtypeStruct((N, D_HID), jnp.float32),
               jax.ShapeDtypeStruct((N, D_OUT), jnp.float32)],
)

_tc2 = pl.pallas_call(
    _tc2_body,
    grid=(TG,),
    in_specs=[
        _whole((NC * N, D_OUT)),
        _whole((NC * N, 16)),
        pl.BlockSpec((TR, D_HID), lambda i: (i, 0)),
        _whole((D_OUT, D_HID)),
        _whole((1, D_OUT)),
    ],
    out_specs=_whole((N, D_OUT)),
    out_shape=jax.ShapeDtypeStruct((N, D_OUT), jnp.float32),
)


def kernel(x, edge_index, Wl1, bl1, Wr1, Wl2, bl2, Wr2):
    src = edge_index[0]
    dst = edge_index[1]
    dst2 = dst.reshape(NT, NCH, CH)

    z128 = jnp.zeros((RCH, D_IN), jnp.float32)
    z16 = jnp.zeros((RCH, 16), jnp.float32)
    s1, cnt = _sc_agg128_cnt(x, src, dst2, z128, z16)
    h, y2 = _tc1(x, s1, s1, cnt, Wl1, bl1.reshape(1, D_HID), Wr1, Wl2)
    s2 = _sc_agg_64(y2, src, dst2)
    out = _tc2(s2, cnt, h, Wr2, bl2.reshape(1, D_OUT))
    return out


# speedup vs baseline: 1.0608x; 1.0400x over previous
import functools

import jax
import jax.numpy as jnp
from jax import lax
from jax.experimental import pallas as pl
from jax.experimental.pallas import tpu as pltpu
from jax.experimental.pallas import tpu_sc as plsc

N = 10000
E = 320000
D_IN = 128
D_HID = 128
D_OUT = 64

NC = 2
NS = 16
NT = NC * NS
EPT = E // NT
CH = 40
NCH = EPT // CH
NB = 5
NRND = NCH // NB
RPT = N // NS
ZR = 125


ZCH = 16
NZCH = N // ZCH
RCH = 2000
NRCH = N // RCH


def _sc_agg64_body(x_hbm, ei4_hbm, z64_hbm, out_hbm,
                   acc, src_v, dst_v, rows_v, gsem, ssem):
    c = lax.axis_index("c")
    s = lax.axis_index("s")
    wid = s * NC + c

    @pl.when(s < NRCH)
    def _():
        r0 = s * RCH
        pltpu.sync_copy(z64_hbm, acc.at[pl.ds(r0, RCH)])

    plsc.subcore_barrier()

    pltpu.sync_copy(ei4_hbm.at[0, wid], src_v)
    pltpu.sync_copy(ei4_hbm.at[1, wid], dst_v)

    def g_desc(j, b):
        return pltpu.make_async_copy(
            x_hbm.at[src_v.at[j]], rows_v.at[b], gsem.at[b])

    def s_desc(j, b):
        return pltpu.make_async_copy(
            rows_v.at[b], acc.at[dst_v.at[j]], ssem.at[b])

    for b in range(NB):
        g_desc(b, b).start()

    def rnd(i, carry):
        j0 = i * NB
        for b in range(NB):
            g_desc(j0 + b, b).wait()
            s_desc(j0 + b, b).start(add=True)
        for b in range(NB):
            s_desc(j0 + b, b).wait()

            @pl.when(i + 1 < NRND)
            def _():
                g_desc(j0 + NB + b, b).start()
        return carry

    lax.fori_loop(0, NRND, rnd, 0)

    plsc.subcore_barrier()

    @pl.when(s < NRCH)
    def _():
        r0 = s * RCH
        pltpu.sync_copy(acc.at[pl.ds(r0, RCH)],
                        out_hbm.at[pl.ds(c * N + r0, RCH)])


_sc_agg_64 = pl.kernel(
    _sc_agg64_body,
    out_type=jax.ShapeDtypeStruct((NC * N, D_OUT), jnp.float32),
    mesh=plsc.VectorSubcoreMesh(core_axis_name="c", subcore_axis_name="s"),
    scratch_types=[
        pltpu.VMEM_SHARED((N, D_OUT), jnp.float32),
        pltpu.VMEM((NCH, CH), jnp.int32),
        pltpu.VMEM((NCH, CH), jnp.int32),
        pltpu.VMEM((NB, CH, D_OUT), jnp.float32),
        pltpu.SemaphoreType.DMA((NB,)),
        pltpu.SemaphoreType.DMA((NB,)),
    ],
    compiler_params=pltpu.CompilerParams(use_tc_tiling_on_sc=False),
)

RND_E = NB * CH


def _sc_agg128_cnt_body(x_hbm, ei4_hbm, z128_hbm, z16_hbm,
                        out_hbm, cnt_hbm,
                        acc, cnt_sh, srcb0, srcb1, dst_v, rows_v, ones_v,
                        gsem, ssem, csem, srcsem0, srcsem1):
    D = D_IN
    c = lax.axis_index("c")
    s = lax.axis_index("s")
    wid = s * NC + c

    ovec = jnp.ones((16,), jnp.float32)

    def orow(r, carry):
        ones_v[r, pl.ds(0, 16)] = ovec
        return carry
    lax.fori_loop(0, CH, orow, 0)

    @pl.when(s < NRCH)
    def _():
        r0 = s * RCH
        pltpu.sync_copy(z128_hbm, acc.at[pl.ds(r0, RCH)])
        pltpu.sync_copy(z16_hbm, cnt_sh.at[pl.ds(r0, RCH)])

    plsc.subcore_barrier()

    pltpu.sync_copy(ei4_hbm.at[1, wid], dst_v)

    srcbs = (srcb0, srcb1)
    srcsems = (srcsem0, srcsem1)

    def srcload(r, par):
        return pltpu.make_async_copy(
            ei4_hbm.at[0, wid, pl.ds(r * NB, NB)], srcbs[par],
            srcsems[par])

    def g_desc(j, b, par):
        return pltpu.make_async_copy(
            x_hbm.at[srcbs[par].at[b]], rows_v.at[b],
            gsem.at[b])

    def s_desc(j, b):
        return pltpu.make_async_copy(
            rows_v.at[b], acc.at[dst_v.at[j]], ssem.at[b])

    def o_desc(j, b):
        return pltpu.make_async_copy(
            ones_v, cnt_sh.at[dst_v.at[j]], csem.at[b])

    srcload(0, 0).start()
    srcload(0, 0).wait()
    srcload(1, 1).start()
    for b in range(NB):
        g_desc(b, b, 0).start()

    def rnd(i, par):
        j0 = i * NB
        for b in range(NB):
            j = j0 + b
            g_desc(j, b, par).wait()
            s_desc(j, b).start(add=True)
            o_desc(j, b).start(add=True)

        @pl.when(i + 2 < NRND)
        def _():
            srcload(i + 2, par).start()

        @pl.when(i + 1 < NRND)
        def _():
            srcload(i + 1, 1 - par).wait()

        for b in range(NB):
            j = j0 + b
            s_desc(j, b).wait()
            o_desc(j, b).wait()

            @pl.when(i + 1 < NRND)
            def _():
                g_desc(j + NB, b, 1 - par).start()

    def rnd_pair(p, carry):
        rnd(2 * p, 0)
        rnd(2 * p + 1, 1)
        return carry

    lax.fori_loop(0, NRND // 2, rnd_pair, 0)

    plsc.subcore_barrier()

    @pl.when(s < NRCH)
    def _():
        r0 = s * RCH
        pltpu.sync_copy(acc.at[pl.ds(r0, RCH)],
                        out_hbm.at[pl.ds(c * N + r0, RCH)])
        pltpu.sync_copy(cnt_sh.at[pl.ds(r0, RCH)],
                        cnt_hbm.at[pl.ds(c * N + r0, RCH)])


_sc_agg128_cnt = pl.kernel(
    _sc_agg128_cnt_body,
    out_type=[jax.ShapeDtypeStruct((NC * N, D_IN), jnp.float32),
              jax.ShapeDtypeStruct((NC * N, 16), jnp.float32)],
    mesh=plsc.VectorSubcoreMesh(core_axis_name="c", subcore_axis_name="s"),
    scratch_types=[
        pltpu.VMEM_SHARED((N, D_IN), jnp.float32),
        pltpu.VMEM_SHARED((N, 16), jnp.float32),
        pltpu.VMEM((NB, CH), jnp.int32),
        pltpu.VMEM((NB, CH), jnp.int32),
        pltpu.VMEM((NCH, CH), jnp.int32),
        pltpu.VMEM((NB, CH, D_IN), jnp.float32),
        pltpu.VMEM((CH, 16), jnp.float32),
        pltpu.SemaphoreType.DMA((NB,)),
        pltpu.SemaphoreType.DMA((NB,)),
        pltpu.SemaphoreType.DMA((NB,)),
        pltpu.SemaphoreType.DMA,
        pltpu.SemaphoreType.DMA,
    ],
    compiler_params=pltpu.CompilerParams(use_tc_tiling_on_sc=False),
)


def _tc1_body(x_ref, s1_ref, cnt_ref, wl1_ref, bl1_ref, wr1_ref, wl2_ref,
              h_ref, y2_ref):
    cnt = jnp.sum(cnt_ref[:N, :] + cnt_ref[N:, :], axis=1) * (1.0 / 16.0)
    ssum = s1_ref[:N, :] + s1_ref[N:, :]
    mean = ssum / jnp.maximum(cnt, 1.0)[:, None]
    dn = (((1,), (1,)), ((), ()))
    h = (lax.dot_general(mean, wl1_ref[...], dn,
                         preferred_element_type=jnp.float32)
         + bl1_ref[...]
         + lax.dot_general(x_ref[...], wr1_ref[...], dn,
                           preferred_element_type=jnp.float32))
    h = jnp.maximum(h, 0.0)
    h_ref[...] = h
    y2_ref[...] = lax.dot_general(h, wl2_ref[...], dn,
                                  preferred_element_type=jnp.float32)


def _tc2_body(s2_ref, cnt_ref, h_ref, wr2_ref, bl2_ref, o_ref):
    cnt = jnp.sum(cnt_ref[:N, :] + cnt_ref[N:, :], axis=1) * (1.0 / 16.0)
    m2 = (s2_ref[:N, :] + s2_ref[N:, :]) / jnp.maximum(cnt, 1.0)[:, None]
    dn = (((1,), (1,)), ((), ()))
    o_ref[...] = (m2 + bl2_ref[...]
                  + lax.dot_general(h_ref[...], wr2_ref[...], dn,
                                    preferred_element_type=jnp.float32))


_tc1 = pl.pallas_call(
    _tc1_body,
    out_shape=[jax.ShapeDtypeStruct((N, D_HID), jnp.float32),
               jax.ShapeDtypeStruct((N, D_OUT), jnp.float32)],
)

_tc2 = pl.pallas_call(
    _tc2_body,
    out_shape=jax.ShapeDtypeStruct((N, D_OUT), jnp.float32),
)


def kernel(x, edge_index, Wl1, bl1, Wr1, Wl2, bl2, Wr2):
    ei4 = edge_index.reshape(2, NT, NCH, CH)

    z128 = jnp.zeros((RCH, D_IN), jnp.float32)
    z16 = jnp.zeros((RCH, 16), jnp.float32)
    z64 = jnp.zeros((RCH, D_OUT), jnp.float32)
    s1, cnt = _sc_agg128_cnt(x, ei4, z128, z16)
    h, y2 = _tc1(x, s1, cnt, Wl1, bl1.reshape(1, D_HID), Wr1, Wl2)
    s2 = _sc_agg_64(y2, ei4, z64)
    out = _tc2(s2, cnt, h, Wr2, bl2.reshape(1, D_OUT))
    return out


# speedup vs baseline: 1.0966x; 1.0337x over previous
import functools

import jax
import jax.numpy as jnp
from jax import lax
from jax.experimental import pallas as pl
from jax.experimental.pallas import tpu as pltpu
from jax.experimental.pallas import tpu_sc as plsc

N = 10000
E = 320000
D_IN = 128
D_HID = 128
D_OUT = 64

NC = 2
NS = 16
NT = NC * NS
EPT = E // NT
CH = 40
NCH = EPT // CH
NB = 5
NRND = NCH // NB
CH2 = 80
NCH2 = EPT // CH2
NRND2 = NCH2 // NB
RPT = N // NS
ZR = 125


ZCH = 16
NZCH = N // ZCH
RCH = 2000
NRCH = N // RCH


def _sc_agg64_body(x_hbm, ei_hbm, z64_hbm, out_hbm,
                   acc, src_v, dst_v, rows_v, gsem, ssem):
    c = lax.axis_index("c")
    s = lax.axis_index("s")
    wid = s * NC + c

    @pl.when(s < NRCH)
    def _():
        r0 = s * RCH
        pltpu.sync_copy(z64_hbm, acc.at[pl.ds(r0, RCH)])

    plsc.subcore_barrier()

    ebase = wid * EPT
    pltpu.sync_copy(ei_hbm.at[0, pl.ds(ebase, EPT)], src_v)
    pltpu.sync_copy(ei_hbm.at[1, pl.ds(ebase, EPT)], dst_v)

    def g_desc(j, b):
        return pltpu.make_async_copy(
            x_hbm.at[src_v.at[pl.ds(j * CH2, CH2)]], rows_v.at[b],
            gsem.at[b])

    def s_desc(j, b):
        return pltpu.make_async_copy(
            rows_v.at[b], acc.at[dst_v.at[pl.ds(j * CH2, CH2)]], ssem.at[b])

    for b in range(NB):
        g_desc(b, b).start()

    def rnd(i, carry):
        j0 = i * NB
        for b in range(NB):
            g_desc(j0 + b, b).wait()
            s_desc(j0 + b, b).start(add=True)
        for b in range(NB):
            s_desc(j0 + b, b).wait()

            @pl.when(i + 1 < NRND2)
            def _():
                g_desc(j0 + NB + b, b).start()
        return carry

    lax.fori_loop(0, NRND2, rnd, 0)

    plsc.subcore_barrier()

    @pl.when(s < NRCH)
    def _():
        r0 = s * RCH
        pltpu.sync_copy(acc.at[pl.ds(r0, RCH)],
                        out_hbm.at[pl.ds(c * N + r0, RCH)])


_sc_agg_64 = pl.kernel(
    _sc_agg64_body,
    out_type=jax.ShapeDtypeStruct((NC * N, D_OUT), jnp.float32),
    mesh=plsc.VectorSubcoreMesh(core_axis_name="c", subcore_axis_name="s"),
    scratch_types=[
        pltpu.VMEM_SHARED((N, D_OUT), jnp.float32),
        pltpu.VMEM((EPT,), jnp.int32),
        pltpu.VMEM((EPT,), jnp.int32),
        pltpu.VMEM((NB, CH2, D_OUT), jnp.float32),
        pltpu.SemaphoreType.DMA((NB,)),
        pltpu.SemaphoreType.DMA((NB,)),
    ],
    compiler_params=pltpu.CompilerParams(use_tc_tiling_on_sc=False),
)

RND_E = NB * CH


def _sc_agg128_cnt_body(x_hbm, ei_hbm, z128_hbm, z16_hbm,
                        out_hbm, cnt_hbm,
                        acc, cnt_sh, srcb0, srcb1, dst_v, rows_v, ones_v,
                        gsem, ssem, csem, srcsem0, srcsem1):
    D = D_IN
    c = lax.axis_index("c")
    s = lax.axis_index("s")
    wid = s * NC + c

    ovec = jnp.ones((16,), jnp.float32)

    def orow(r, carry):
        ones_v[r, pl.ds(0, 16)] = ovec
        return carry
    lax.fori_loop(0, CH, orow, 0)

    @pl.when(s < NRCH)
    def _():
        r0 = s * RCH
        pltpu.sync_copy(z128_hbm, acc.at[pl.ds(r0, RCH)])
        pltpu.sync_copy(z16_hbm, cnt_sh.at[pl.ds(r0, RCH)])

    plsc.subcore_barrier()

    ebase = wid * EPT
    pltpu.sync_copy(ei_hbm.at[1, pl.ds(ebase, EPT)], dst_v)

    srcbs = (srcb0, srcb1)
    srcsems = (srcsem0, srcsem1)

    def srcload(r, par):
        return pltpu.make_async_copy(
            ei_hbm.at[0, pl.ds(ebase + r * RND_E, RND_E)], srcbs[par],
            srcsems[par])

    def g_desc(j, b, par):
        return pltpu.make_async_copy(
            x_hbm.at[srcbs[par].at[pl.ds(b * CH, CH)]], rows_v.at[b],
            gsem.at[b])

    def s_desc(j, b):
        return pltpu.make_async_copy(
            rows_v.at[b], acc.at[dst_v.at[pl.ds(j * CH, CH)]], ssem.at[b])

    def o_desc(j, b):
        return pltpu.make_async_copy(
            ones_v, cnt_sh.at[dst_v.at[pl.ds(j * CH, CH)]], csem.at[b])

    srcload(0, 0).start()
    srcload(0, 0).wait()
    srcload(1, 1).start()
    for b in range(NB):
        g_desc(b, b, 0).start()

    def rnd(i, par):
        j0 = i * NB
        for b in range(NB):
            j = j0 + b
            g_desc(j, b, par).wait()
            s_desc(j, b).start(add=True)
            o_desc(j, b).start(add=True)

        @pl.when(i + 2 < NRND)
        def _():
            srcload(i + 2, par).start()

        @pl.when(i + 1 < NRND)
        def _():
            srcload(i + 1, 1 - par).wait()

        for b in range(NB):
            j = j0 + b
            s_desc(j, b).wait()
            o_desc(j, b).wait()

            @pl.when(i + 1 < NRND)
            def _():
                g_desc(j + NB, b, 1 - par).start()

    def rnd_pair(p, carry):
        rnd(2 * p, 0)
        rnd(2 * p + 1, 1)
        return carry

    lax.fori_loop(0, NRND // 2, rnd_pair, 0)

    plsc.subcore_barrier()

    @pl.when(s < NRCH)
    def _():
        r0 = s * RCH
        pltpu.sync_copy(acc.at[pl.ds(r0, RCH)],
                        out_hbm.at[pl.ds(c * N + r0, RCH)])
        pltpu.sync_copy(cnt_sh.at[pl.ds(r0, RCH)],
                        cnt_hbm.at[pl.ds(c * N + r0, RCH)])


_sc_agg128_cnt = pl.kernel(
    _sc_agg128_cnt_body,
    out_type=[jax.ShapeDtypeStruct((NC * N, D_IN), jnp.float32),
              jax.ShapeDtypeStruct((NC * N, 16), jnp.float32)],
    mesh=plsc.VectorSubcoreMesh(core_axis_name="c", subcore_axis_name="s"),
    scratch_types=[
        pltpu.VMEM_SHARED((N, D_IN), jnp.float32),
        pltpu.VMEM_SHARED((N, 16), jnp.float32),
        pltpu.VMEM((RND_E,), jnp.int32),
        pltpu.VMEM((RND_E,), jnp.int32),
        pltpu.VMEM((EPT,), jnp.int32),
        pltpu.VMEM((NB, CH, D_IN), jnp.float32),
        pltpu.VMEM((CH, 16), jnp.float32),
        pltpu.SemaphoreType.DMA((NB,)),
        pltpu.SemaphoreType.DMA((NB,)),
        pltpu.SemaphoreType.DMA((NB,)),
        pltpu.SemaphoreType.DMA,
        pltpu.SemaphoreType.DMA,
    ],
    compiler_params=pltpu.CompilerParams(use_tc_tiling_on_sc=False),
)


def _tc1_body(x_ref, s1_ref, cnt_ref, wl1_ref, bl1_ref, wr1_ref, wl2_ref,
              h_ref, y2_ref):
    cnt = jnp.sum(cnt_ref[:N, :] + cnt_ref[N:, :], axis=1) * (1.0 / 16.0)
    ssum = s1_ref[:N, :] + s1_ref[N:, :]
    mean = ssum / jnp.maximum(cnt, 1.0)[:, None]
    dn = (((1,), (1,)), ((), ()))
    h = (lax.dot_general(mean, wl1_ref[...], dn,
                         preferred_element_type=jnp.float32)
         + bl1_ref[...]
         + lax.dot_general(x_ref[...], wr1_ref[...], dn,
                           preferred_element_type=jnp.float32))
    h = jnp.maximum(h, 0.0)
    h_ref[...] = h
    y2_ref[...] = lax.dot_general(h, wl2_ref[...], dn,
                                  preferred_element_type=jnp.float32)


def _tc2_body(s2_ref, cnt_ref, h_ref, wr2_ref, bl2_ref, o_ref):
    cnt = jnp.sum(cnt_ref[:N, :] + cnt_ref[N:, :], axis=1) * (1.0 / 16.0)
    m2 = (s2_ref[:N, :] + s2_ref[N:, :]) / jnp.maximum(cnt, 1.0)[:, None]
    dn = (((1,), (1,)), ((), ()))
    o_ref[...] = (m2 + bl2_ref[...]
                  + lax.dot_general(h_ref[...], wr2_ref[...], dn,
                                    preferred_element_type=jnp.float32))


_tc1 = pl.pallas_call(
    _tc1_body,
    out_shape=[jax.ShapeDtypeStruct((N, D_HID), jnp.float32),
               jax.ShapeDtypeStruct((N, D_OUT), jnp.float32)],
)

_tc2 = pl.pallas_call(
    _tc2_body,
    out_shape=jax.ShapeDtypeStruct((N, D_OUT), jnp.float32),
)


def kernel(x, edge_index, Wl1, bl1, Wr1, Wl2, bl2, Wr2):
    z128 = jnp.zeros((RCH, D_IN), jnp.float32)
    z16 = jnp.zeros((RCH, 16), jnp.float32)
    z64 = jnp.zeros((RCH, D_OUT), jnp.float32)
    s1, cnt = _sc_agg128_cnt(x, edge_index, z128, z16)
    h, y2 = _tc1(x, s1, cnt, Wl1, bl1.reshape(1, D_HID), Wr1, Wl2)
    s2 = _sc_agg_64(y2, edge_index, z64)
    out = _tc2(s2, cnt, h, Wr2, bl2.reshape(1, D_OUT))
    return out


# speedup vs baseline: 1.1972x; 1.0918x over previous
import functools

import jax
import jax.numpy as jnp
from jax import lax
from jax.experimental import pallas as pl
from jax.experimental.pallas import tpu as pltpu
from jax.experimental.pallas import tpu_sc as plsc

N = 10000
E = 320000
D_IN = 128
D_HID = 128
D_OUT = 64

NC = 2
NS = 16
NT = NC * NS
EPT = E // NT
CH = 40
NCH = EPT // CH
NB = 5
NRND = NCH // NB
CH2 = 80
NCH2 = EPT // CH2
NRND2 = NCH2 // NB
RPT = N // NS
ZR = 125


ZCH = 16
NZCH = N // ZCH
RCH = 2000
NRCH = N // RCH


def _sc_agg64_body(x_hbm, ei_hbm, z64_hbm, out_hbm,
                   acc, src_v, dst_v, rows_v, gsem, ssem):
    c = lax.axis_index("c")
    s = lax.axis_index("s")
    wid = s * NC + c

    @pl.when(s < NRCH)
    def _():
        r0 = s * RCH
        pltpu.sync_copy(z64_hbm, acc.at[pl.ds(r0, RCH)])

    plsc.subcore_barrier()

    ebase = wid * EPT
    pltpu.sync_copy(ei_hbm.at[0, pl.ds(ebase, EPT)], src_v)
    pltpu.sync_copy(ei_hbm.at[1, pl.ds(ebase, EPT)], dst_v)

    def g_desc(j, b):
        return pltpu.make_async_copy(
            x_hbm.at[src_v.at[pl.ds(j * CH2, CH2)]], rows_v.at[b],
            gsem.at[b])

    def s_desc(j, b):
        return pltpu.make_async_copy(
            rows_v.at[b], acc.at[dst_v.at[pl.ds(j * CH2, CH2)]], ssem.at[b])

    for b in range(NB):
        g_desc(b, b).start()

    def rnd(i, carry):
        j0 = i * NB
        for b in range(NB):
            g_desc(j0 + b, b).wait()
            s_desc(j0 + b, b).start(add=True)
        for b in range(NB):
            s_desc(j0 + b, b).wait()

            @pl.when(i + 1 < NRND2)
            def _():
                g_desc(j0 + NB + b, b).start()
        return carry

    lax.fori_loop(0, NRND2, rnd, 0)

    plsc.subcore_barrier()

    @pl.when(s < NRCH)
    def _():
        r0 = s * RCH
        pltpu.sync_copy(acc.at[pl.ds(r0, RCH)],
                        out_hbm.at[pl.ds(c * N + r0, RCH)])


_sc_agg_64 = pl.kernel(
    _sc_agg64_body,
    out_type=jax.ShapeDtypeStruct((NC * N, D_OUT), jnp.bfloat16),
    mesh=plsc.VectorSubcoreMesh(core_axis_name="c", subcore_axis_name="s"),
    scratch_types=[
        pltpu.VMEM_SHARED((N, D_OUT), jnp.bfloat16),
        pltpu.VMEM((EPT,), jnp.int32),
        pltpu.VMEM((EPT,), jnp.int32),
        pltpu.VMEM((NB, CH2, D_OUT), jnp.bfloat16),
        pltpu.SemaphoreType.DMA((NB,)),
        pltpu.SemaphoreType.DMA((NB,)),
    ],
    compiler_params=pltpu.CompilerParams(use_tc_tiling_on_sc=False),
)

RND_E = NB * CH


def _sc_agg128_cnt_body(x_hbm, ei_hbm, z128_hbm, z16_hbm,
                        out_hbm, cnt_hbm,
                        acc, cnt_sh, srcb0, srcb1, dst_v, rows_v, ones_v,
                        gsem, ssem, csem, srcsem0, srcsem1):
    D = D_IN
    c = lax.axis_index("c")
    s = lax.axis_index("s")
    wid = s * NC + c

    ovec = jnp.ones((16,), jnp.float32)

    def orow(r, carry):
        ones_v[r, pl.ds(0, 16)] = ovec
        return carry
    lax.fori_loop(0, CH, orow, 0)

    @pl.when(s < NRCH)
    def _():
        r0 = s * RCH
        pltpu.sync_copy(z128_hbm, acc.at[pl.ds(r0, RCH)])
        pltpu.sync_copy(z16_hbm, cnt_sh.at[pl.ds(r0, RCH)])

    plsc.subcore_barrier()

    ebase = wid * EPT
    pltpu.sync_copy(ei_hbm.at[1, pl.ds(ebase, EPT)], dst_v)

    srcbs = (srcb0, srcb1)
    srcsems = (srcsem0, srcsem1)

    def srcload(r, par):
        return pltpu.make_async_copy(
            ei_hbm.at[0, pl.ds(ebase + r * RND_E, RND_E)], srcbs[par],
            srcsems[par])

    def g_desc(j, b, par):
        return pltpu.make_async_copy(
            x_hbm.at[srcbs[par].at[pl.ds(b * CH, CH)]], rows_v.at[b],
            gsem.at[b])

    def s_desc(j, b):
        return pltpu.make_async_copy(
            rows_v.at[b], acc.at[dst_v.at[pl.ds(j * CH, CH)]], ssem.at[b])

    def o_desc(j, b):
        return pltpu.make_async_copy(
            ones_v, cnt_sh.at[dst_v.at[pl.ds(j * CH, CH)]], csem.at[b])

    srcload(0, 0).start()
    srcload(0, 0).wait()
    srcload(1, 1).start()
    for b in range(NB):
        g_desc(b, b, 0).start()

    def rnd(i, par):
        j0 = i * NB
        for b in range(NB):
            j = j0 + b
            g_desc(j, b, par).wait()
            s_desc(j, b).start(add=True)
            o_desc(j, b).start(add=True)

        @pl.when(i + 2 < NRND)
        def _():
            srcload(i + 2, par).start()

        @pl.when(i + 1 < NRND)
        def _():
            srcload(i + 1, 1 - par).wait()

        for b in range(NB):
            j = j0 + b
            s_desc(j, b).wait()
            o_desc(j, b).wait()

            @pl.when(i + 1 < NRND)
            def _():
                g_desc(j + NB, b, 1 - par).start()

    def rnd_pair(p, carry):
        rnd(2 * p, 0)
        rnd(2 * p + 1, 1)
        return carry

    lax.fori_loop(0, NRND // 2, rnd_pair, 0)

    plsc.subcore_barrier()

    @pl.when(s < NRCH)
    def _():
        r0 = s * RCH
        pltpu.sync_copy(acc.at[pl.ds(r0, RCH)],
                        out_hbm.at[pl.ds(c * N + r0, RCH)])
        pltpu.sync_copy(cnt_sh.at[pl.ds(r0, RCH)],
                        cnt_hbm.at[pl.ds(c * N + r0, RCH)])


_sc_agg128_cnt = pl.kernel(
    _sc_agg128_cnt_body,
    out_type=[jax.ShapeDtypeStruct((NC * N, D_IN), jnp.float32),
              jax.ShapeDtypeStruct((NC * N, 16), jnp.float32)],
    mesh=plsc.VectorSubcoreMesh(core_axis_name="c", subcore_axis_name="s"),
    scratch_types=[
        pltpu.VMEM_SHARED((N, D_IN), jnp.float32),
        pltpu.VMEM_SHARED((N, 16), jnp.float32),
        pltpu.VMEM((RND_E,), jnp.int32),
        pltpu.VMEM((RND_E,), jnp.int32),
        pltpu.VMEM((EPT,), jnp.int32),
        pltpu.VMEM((NB, CH, D_IN), jnp.float32),
        pltpu.VMEM((CH, 16), jnp.float32),
        pltpu.SemaphoreType.DMA((NB,)),
        pltpu.SemaphoreType.DMA((NB,)),
        pltpu.SemaphoreType.DMA((NB,)),
        pltpu.SemaphoreType.DMA,
        pltpu.SemaphoreType.DMA,
    ],
    compiler_params=pltpu.CompilerParams(use_tc_tiling_on_sc=False),
)


def _tc1_body(x_ref, s1_ref, cnt_ref, wl1_ref, bl1_ref, wr1_ref, wl2_ref,
              h_ref, y2_ref):
    cnt = jnp.sum(cnt_ref[:N, :] + cnt_ref[N:, :], axis=1) * (1.0 / 16.0)
    ssum = s1_ref[:N, :] + s1_ref[N:, :]
    mean = ssum / jnp.maximum(cnt, 1.0)[:, None]
    dn = (((1,), (1,)), ((), ()))
    h = (lax.dot_general(mean, wl1_ref[...], dn,
                         preferred_element_type=jnp.float32)
         + bl1_ref[...]
         + lax.dot_general(x_ref[...], wr1_ref[...], dn,
                           preferred_element_type=jnp.float32))
    h = jnp.maximum(h, 0.0)
    h_ref[...] = h
    y2_ref[...] = lax.dot_general(h, wl2_ref[...], dn,
                                  preferred_element_type=jnp.float32
                                  ).astype(jnp.bfloat16)


def _tc2_body(s2_ref, cnt_ref, h_ref, wr2_ref, bl2_ref, o_ref):
    cnt = jnp.sum(cnt_ref[:N, :] + cnt_ref[N:, :], axis=1) * (1.0 / 16.0)
    s2f = (s2_ref[:N, :].astype(jnp.float32)
           + s2_ref[N:, :].astype(jnp.float32))
    m2 = s2f / jnp.maximum(cnt, 1.0)[:, None]
    dn = (((1,), (1,)), ((), ()))
    o_ref[...] = (m2 + bl2_ref[...]
                  + lax.dot_general(h_ref[...], wr2_ref[...], dn,
                                    preferred_element_type=jnp.float32))


_tc1 = pl.pallas_call(
    _tc1_body,
    out_shape=[jax.ShapeDtypeStruct((N, D_HID), jnp.float32),
               jax.ShapeDtypeStruct((N, D_OUT), jnp.bfloat16)],
)

_tc2 = pl.pallas_call(
    _tc2_body,
    out_shape=jax.ShapeDtypeStruct((N, D_OUT), jnp.float32),
)


def kernel(x, edge_index, Wl1, bl1, Wr1, Wl2, bl2, Wr2):
    z128 = jnp.zeros((RCH, D_IN), jnp.float32)
    z16 = jnp.zeros((RCH, 16), jnp.float32)
    z64 = jnp.zeros((RCH, D_OUT), jnp.bfloat16)
    s1, cnt = _sc_agg128_cnt(x, edge_index, z128, z16)
    h, y2 = _tc1(x, s1, cnt, Wl1, bl1.reshape(1, D_HID), Wr1, Wl2)
    s2 = _sc_agg_64(y2, edge_index, z64)
    out = _tc2(s2, cnt, h, Wr2, bl2.reshape(1, D_OUT))
    return out
